# baseline (speedup 1.0000x reference)
import functools

import jax
import jax.numpy as jnp
from jax import lax
from jax.experimental import pallas as pl
from jax.experimental.pallas import tpu as pltpu
from jax.experimental.pallas import tpu_sc as plsc

_WEIGHT = 0.1
_MOMENT = 0.9
_WARM_UP = 1.0 / 1000.0
_COOL_DOWN = 1.0 / 100000.0
_MAX_ITEMS = 1000000
_ITERATIONS = 500.0

_ROWS = _MAX_ITEMS + 1
_DIM = 128
_BATCH = 16384

_NC = 2
_NS = 16
_LANES = 16
_SENT = -1

_PER_TILE = _BATCH // _NS
_CHUNK = 128
_NCHUNK = _PER_TILE // _CHUNK
_GROUPS = _CHUNK // _LANES

_COPY_BLK = 8192


def _copy_body(src_ref, dst_ref):
    dst_ref[...] = src_ref[...]


def _pallas_copy(history):
    n_blocks = (_ROWS + _COPY_BLK - 1) // _COPY_BLK
    return pl.pallas_call(
        _copy_body,
        grid=(n_blocks,),
        in_specs=[pl.BlockSpec((_COPY_BLK, _DIM), lambda i: (i, 0))],
        out_specs=pl.BlockSpec((_COPY_BLK, _DIM), lambda i: (i, 0)),
        out_shape=jax.ShapeDtypeStruct((_ROWS, _DIM), jnp.float32),
    )(history)


def _sc_body(
    act_hbm,
    idx_hbm,
    hist_hbm,
    out_ref,
    loss_hbm,
    tag_sp,
    cnt_sm,
    sidx,
    gidx,
    lgidx,
    rgidx,
    tgot,
    bval,
    act_c,
    row_c,
    val_c,
    acc_v,
):
    c = lax.axis_index("c")
    s = lax.axis_index("s")
    slot0 = s * _PER_TILE

    pltpu.sync_copy(idx_hbm.at[pl.ds(slot0, _PER_TILE)], sidx)

    lanes = lax.iota(jnp.int32, _LANES)

    def _build(i, _):
        v = sidx[pl.ds(i * _LANES, _LANES)]
        active = (
            (v != 0)
            & (v < _MAX_ITEMS)
            & ((v & 1) == c)
        )
        gidx[pl.ds(i * _LANES, _LANES)] = jnp.where(active, v, _SENT)
        bval[pl.ds(i * _LANES, _LANES)] = slot0 + i * _LANES + lanes
        return 0

    lax.fori_loop(0, _PER_TILE // _LANES, _build, 0)

    pltpu.sync_copy(bval, tag_sp.at[plsc.Indices(gidx, ignored_value=_SENT)])
    plsc.subcore_barrier()
    pltpu.sync_copy(tag_sp.at[plsc.Indices(gidx, ignored_value=_SENT)], tgot)

    def _elect(i, _):
        sl = pl.ds(i * _LANES, _LANES)
        g = gidx[sl]
        lead = (g != _SENT) & (tgot[sl] == bval[sl])
        lgidx[sl] = jnp.where(lead, g, _SENT)
        rgidx[sl] = jnp.where(lead, _SENT, g)
        return 0

    lax.fori_loop(0, _PER_TILE // _LANES, _elect, 0)

    acc = jnp.zeros((_LANES,), jnp.float32)
    for j in range(_NCHUNK):
        base = j * _CHUNK
        pltpu.sync_copy(act_hbm.at[pl.ds(slot0 + base, _CHUNK)], act_c)
        pltpu.sync_copy(
            hist_hbm.at[
                plsc.Indices(gidx.at[pl.ds(base, _CHUNK)], ignored_value=_SENT)
            ],
            row_c,
        )
        for g in range(_GROUPS):
            rows = g * _LANES + lanes
            am = gidx[pl.ds(base + g * _LANES, _LANES)] != _SENT

            def _cols(col, a, rows=rows, am=am):
                cols = jnp.full((_LANES,), col, jnp.int32)
                h = plsc.load_gather(row_c, [rows, cols])
                av = plsc.load_gather(act_c, [rows, cols])
                d = jnp.where(am, h - av, 0.0)
                v = h - (1.0 - _MOMENT) * d
                plsc.store_scatter(val_c, [rows, cols], v)
                return a + d * d

            acc = lax.fori_loop(0, _DIM, _cols, acc)
        pltpu.sync_copy(
            val_c,
            out_ref.at[
                plsc.Indices(lgidx.at[pl.ds(base, _CHUNK)], ignored_value=_SENT)
            ],
        )

    acc_v[...] = acc
    wid = c * _NS + s
    pltpu.sync_copy(acc_v, loss_hbm.at[wid])

    def _count_remaining():
        def _cnt(i, a):
            r = rgidx[pl.ds(i * _LANES, _LANES)]
            return a + jnp.where(r != _SENT, 1, 0)

        local = jnp.sum(
            lax.fori_loop(
                0, _PER_TILE // _LANES, _cnt, jnp.zeros((_LANES,), jnp.int32)
            )
        )

        plsc.subcore_barrier()

        @pl.when(s == 0)
        def _():
            cnt_sm[0] = 0

        plsc.subcore_barrier()
        plsc.fetch_and_add(cnt_sm, local, subcore_id=0)
        plsc.subcore_barrier()
        return plsc.fetch_and_add(cnt_sm, 0, subcore_id=0)

    def _round(total):
        pltpu.sync_copy(
            bval, tag_sp.at[plsc.Indices(rgidx, ignored_value=_SENT)]
        )
        plsc.subcore_barrier()
        pltpu.sync_copy(
            tag_sp.at[plsc.Indices(rgidx, ignored_value=_SENT)], tgot
        )

        def _elect2(i, _):
            sl = pl.ds(i * _LANES, _LANES)
            r = rgidx[sl]
            lead = (r != _SENT) & (tgot[sl] == bval[sl])
            lgidx[sl] = jnp.where(lead, r, _SENT)
            rgidx[sl] = jnp.where(lead, _SENT, r)
            return 0

        lax.fori_loop(0, _PER_TILE // _LANES, _elect2, 0)

        for j in range(_NCHUNK):
            base = j * _CHUNK
            lsl = plsc.Indices(
                lgidx.at[pl.ds(base, _CHUNK)], ignored_value=_SENT
            )
            pltpu.sync_copy(act_hbm.at[pl.ds(slot0 + base, _CHUNK)], act_c)
            pltpu.sync_copy(hist_hbm.at[lsl], row_c)
            for g in range(_GROUPS):
                rows = g * _LANES + lanes

                def _cols2(col, _, rows=rows):
                    cols = jnp.full((_LANES,), col, jnp.int32)
                    h = plsc.load_gather(row_c, [rows, cols])
                    av = plsc.load_gather(act_c, [rows, cols])
                    plsc.store_scatter(
                        val_c, [rows, cols], (1.0 - _MOMENT) * (h - av)
                    )
                    return 0

                lax.fori_loop(0, _DIM, _cols2, 0)
            pltpu.sync_copy(out_ref.at[lsl], row_c)
            for g in range(_GROUPS):
                rows = g * _LANES + lanes

                def _cols3(col, _, rows=rows):
                    cols = jnp.full((_LANES,), col, jnp.int32)
                    cur = plsc.load_gather(row_c, [rows, cols])
                    dd = plsc.load_gather(val_c, [rows, cols])
                    plsc.store_scatter(val_c, [rows, cols], cur - dd)
                    return 0

                lax.fori_loop(0, _DIM, _cols3, 0)
            pltpu.sync_copy(val_c, out_ref.at[lsl])

        return _count_remaining()

    total0 = _count_remaining()
    lax.while_loop(lambda t: t > 0, _round, total0)


_sc_kernel = pl.kernel(
    _sc_body,
    out_type=jax.ShapeDtypeStruct((_NC * _NS, _LANES), jnp.float32),
    mesh=plsc.VectorSubcoreMesh(
        core_axis_name="c", subcore_axis_name="s", num_cores=_NC,
        num_subcores=_NS,
    ),
    compiler_params=pltpu.CompilerParams(needs_layout_passes=False),
    scratch_types=[
        pltpu.VMEM_SHARED((_ROWS,), jnp.int32),
        pltpu.SMEM((1,), jnp.int32),
        pltpu.VMEM((_PER_TILE,), jnp.int32),
        pltpu.VMEM((_PER_TILE,), jnp.int32),
        pltpu.VMEM((_PER_TILE,), jnp.int32),
        pltpu.VMEM((_PER_TILE,), jnp.int32),
        pltpu.VMEM((_PER_TILE,), jnp.int32),
        pltpu.VMEM((_PER_TILE,), jnp.int32),
        pltpu.VMEM((_CHUNK, _DIM), jnp.float32),
        pltpu.VMEM((_CHUNK, _DIM), jnp.float32),
        pltpu.VMEM((_CHUNK, _DIM), jnp.float32),
        pltpu.VMEM((_LANES,), jnp.float32),
    ],
)


def kernel(activations, samples, history):
    idx = jnp.minimum(samples, _MAX_ITEMS).astype(jnp.int32).reshape(-1)
    copy = _pallas_copy(history)
    out_ref = jax.new_ref(copy)
    parts = _sc_kernel(activations, idx, history, out_ref)
    warm_up = _WARM_UP * _ITERATIONS
    cool_down = _COOL_DOWN * _ITERATIONS
    loss = (
        _WEIGHT
        * (jnp.sum(parts) / (_BATCH * _DIM))
        * warm_up
        / (1.0 + warm_up)
        / (1.0 + cool_down)
    )
    new_history = out_ref[...]
    return (activations, loss, new_history)


# speedup vs baseline: 3.6408x; 3.6408x over previous
import jax
import jax.numpy as jnp
from jax import lax
from jax.experimental import pallas as pl
from jax.experimental.pallas import tpu as pltpu
from jax.experimental.pallas import tpu_sc as plsc

_WEIGHT = 0.1
_MOMENT = 0.9
_WARM_UP = 1.0 / 1000.0
_COOL_DOWN = 1.0 / 100000.0
_MAX_ITEMS = 1000000
_ITERATIONS = 500.0

_ROWS = _MAX_ITEMS + 1
_DIM = 128
_BATCH = 16384
_NVREG = _DIM // 16

_NC = 2
_NS = 16
_LANES = 16
_SENT = -1

_PER_TILE = _BATCH // _NS
_CHUNK = 128
_NCHUNK = _PER_TILE // _CHUNK

_COPY_BLK = 8192


def _copy_body(src_ref, dst_ref):
    dst_ref[...] = src_ref[...]


def _pallas_copy(history):
    n_blocks = (_ROWS + _COPY_BLK - 1) // _COPY_BLK
    return pl.pallas_call(
        _copy_body,
        grid=(n_blocks,),
        in_specs=[pl.BlockSpec((_COPY_BLK, _DIM), lambda i: (i, 0))],
        out_specs=pl.BlockSpec((_COPY_BLK, _DIM), lambda i: (i, 0)),
        out_shape=jax.ShapeDtypeStruct((_ROWS, _DIM), jnp.float32),
    )(history)


def _sc_body(
    act_hbm,
    idx_hbm,
    hist_hbm,
    out_ref,
    loss_hbm,
    tag_sp,
    cnt_sm,
    sidx,
    gidx,
    lgidx,
    tgot,
    bval,
    rem_pos,
    act_c,
    row_c,
    val_c,
    b16,
    t16,
    tgix,
    lgix,
    asix,
    h16,
    a16,
    c16,
    v16,
    acc_v,
):
    c = lax.axis_index("c")
    s = lax.axis_index("s")
    slot0 = s * _PER_TILE
    lanes = lax.iota(jnp.int32, _LANES)

    pltpu.sync_copy(idx_hbm.at[pl.ds(slot0, _PER_TILE)], sidx)

    def _build(i, _):
        sl = pl.ds(i * _LANES, _LANES)
        v = sidx[sl]
        active = (v != 0) & (v < _MAX_ITEMS) & ((v & 1) == c)
        gidx[sl] = jnp.where(active, v, _SENT)
        bval[sl] = slot0 + i * _LANES + lanes
        return 0

    lax.fori_loop(0, _PER_TILE // _LANES, _build, 0)

    tag_ix = plsc.Indices(gidx, ignored_value=_SENT)
    pltpu.sync_copy(bval, tag_sp.at[tag_ix])
    plsc.subcore_barrier()
    pltpu.sync_copy(tag_sp.at[tag_ix], tgot)

    def _elect(i, off):
        sl = pl.ds(i * _LANES, _LANES)
        g = gidx[sl]
        act = g != _SENT
        lead = act & (tgot[sl] == bval[sl])
        lgidx[sl] = jnp.where(lead, g, _SENT)
        rem = jnp.where(act & ~lead, 1, 0)
        dst = jnp.maximum(off + plsc.cumsum(rem) - 1, 0)
        plsc.store_scatter(
            rem_pos, [dst], i * _LANES + lanes, mask=rem != 0
        )
        return off + jnp.sum(rem)

    n_rem = lax.fori_loop(0, _PER_TILE // _LANES, _elect, 0)

    acc = jnp.zeros((_LANES,), jnp.float32)
    for j in range(_NCHUNK):
        base = j * _CHUNK
        pltpu.sync_copy(act_hbm.at[pl.ds(slot0 + base, _CHUNK)], act_c)
        pltpu.sync_copy(
            hist_hbm.at[
                plsc.Indices(gidx.at[pl.ds(base, _CHUNK)], ignored_value=_SENT)
            ],
            row_c,
        )

        def _slots(i, a, base=base):
            isp = jnp.full((_LANES,), i, jnp.int32)
            am = (
                plsc.load_gather(gidx, [jnp.full((_LANES,), base, jnp.int32) + i])
                != _SENT
            )
            for r in range(_NVREG):
                cols = r * _LANES + lanes
                h = plsc.load_gather(row_c, [isp, cols])
                av = plsc.load_gather(act_c, [isp, cols])
                d = jnp.where(am, h - av, 0.0)
                a = a + d * d
                plsc.store_scatter(val_c, [isp, cols], h - (1.0 - _MOMENT) * d)
            return a

        acc = lax.fori_loop(0, _CHUNK, _slots, acc)
        pltpu.sync_copy(
            val_c,
            out_ref.at[
                plsc.Indices(lgidx.at[pl.ds(base, _CHUNK)], ignored_value=_SENT)
            ],
        )

    acc_v[...] = acc
    pltpu.sync_copy(acc_v, loss_hbm.at[c * _NS + s])

    def _count(local):
        plsc.subcore_barrier()

        @pl.when(s == 0)
        def _():
            cnt_sm[0] = 0

        plsc.subcore_barrier()
        plsc.fetch_and_add(cnt_sm, local, subcore_id=0)
        plsc.subcore_barrier()
        return plsc.fetch_and_add(cnt_sm, 0, subcore_id=0)

    def _round(carry):
        _, n_rem = carry
        ng = (n_rem + _LANES - 1) // _LANES

        def _scat(k, _):
            lm = k * _LANES + lanes < n_rem
            pos = jnp.where(lm, rem_pos[pl.ds(k * _LANES, _LANES)], 0)
            g = plsc.load_gather(gidx, [pos])
            tgix[...] = jnp.where(lm, g, _SENT)
            b16[...] = slot0 + pos
            pltpu.sync_copy(
                b16, tag_sp.at[plsc.Indices(tgix, ignored_value=_SENT)]
            )
            return 0

        lax.fori_loop(0, ng, _scat, 0)
        plsc.subcore_barrier()

        def _proc(k, off):
            lm = k * _LANES + lanes < n_rem
            pos = jnp.where(lm, rem_pos[pl.ds(k * _LANES, _LANES)], 0)
            g = plsc.load_gather(gidx, [pos])
            b = slot0 + pos
            tgix[...] = jnp.where(lm, g, _SENT)
            pltpu.sync_copy(
                tag_sp.at[plsc.Indices(tgix, ignored_value=_SENT)], t16
            )
            lead = lm & (t16[...] == b)
            lgix[...] = jnp.where(lead, g, _SENT)
            asix[...] = jnp.where(lead, b, _SENT)
            lix = plsc.Indices(lgix, ignored_value=_SENT)
            pltpu.sync_copy(hist_hbm.at[lix], h16)
            pltpu.sync_copy(
                act_hbm.at[plsc.Indices(asix, ignored_value=_SENT)], a16
            )
            pltpu.sync_copy(out_ref.at[lix], c16)

            def _rows(i, _):
                isp = jnp.full((_LANES,), i, jnp.int32)
                for r in range(_NVREG):
                    cols = r * _LANES + lanes
                    hh = plsc.load_gather(h16, [isp, cols])
                    aa = plsc.load_gather(a16, [isp, cols])
                    cc = plsc.load_gather(c16, [isp, cols])
                    plsc.store_scatter(
                        v16, [isp, cols], cc - (1.0 - _MOMENT) * (hh - aa)
                    )
                return 0

            lax.fori_loop(0, _LANES, _rows, 0)
            pltpu.sync_copy(v16, out_ref.at[lix])

            keep = jnp.where(lm & ~lead, 1, 0)
            dst = jnp.maximum(off + plsc.cumsum(keep) - 1, 0)
            plsc.store_scatter(rem_pos, [dst], pos, mask=keep != 0)
            return off + jnp.sum(keep)

        n_rem = lax.fori_loop(0, ng, _proc, 0)
        return _count(n_rem), n_rem

    total = _count(n_rem)
    lax.while_loop(lambda t: t[0] > 0, _round, (total, n_rem))


_sc_kernel = pl.kernel(
    _sc_body,
    out_type=jax.ShapeDtypeStruct((_NC * _NS, _LANES), jnp.float32),
    mesh=plsc.VectorSubcoreMesh(
        core_axis_name="c", subcore_axis_name="s", num_cores=_NC,
        num_subcores=_NS,
    ),
    compiler_params=pltpu.CompilerParams(needs_layout_passes=False),
    scratch_types=[
        pltpu.VMEM_SHARED((_ROWS,), jnp.int32),
        pltpu.SMEM((1,), jnp.int32),
        pltpu.VMEM((_PER_TILE,), jnp.int32),
        pltpu.VMEM((_PER_TILE,), jnp.int32),
        pltpu.VMEM((_PER_TILE,), jnp.int32),
        pltpu.VMEM((_PER_TILE,), jnp.int32),
        pltpu.VMEM((_PER_TILE,), jnp.int32),
        pltpu.VMEM((_PER_TILE + _LANES,), jnp.int32),
        pltpu.VMEM((_CHUNK, _DIM), jnp.float32),
        pltpu.VMEM((_CHUNK, _DIM), jnp.float32),
        pltpu.VMEM((_CHUNK, _DIM), jnp.float32),
        pltpu.VMEM((_LANES,), jnp.int32),
        pltpu.VMEM((_LANES,), jnp.int32),
        pltpu.VMEM((_LANES,), jnp.int32),
        pltpu.VMEM((_LANES,), jnp.int32),
        pltpu.VMEM((_LANES,), jnp.int32),
        pltpu.VMEM((_LANES, _DIM), jnp.float32),
        pltpu.VMEM((_LANES, _DIM), jnp.float32),
        pltpu.VMEM((_LANES, _DIM), jnp.float32),
        pltpu.VMEM((_LANES, _DIM), jnp.float32),
        pltpu.VMEM((_LANES,), jnp.float32),
    ],
)


def kernel(activations, samples, history):
    idx = jnp.minimum(samples, _MAX_ITEMS).astype(jnp.int32).reshape(-1)
    copy = _pallas_copy(history)
    out_ref = jax.new_ref(copy)
    parts = _sc_kernel(activations, idx, history, out_ref)
    warm_up = _WARM_UP * _ITERATIONS
    cool_down = _COOL_DOWN * _ITERATIONS
    loss = (
        _WEIGHT
        * (jnp.sum(parts) / (_BATCH * _DIM))
        * warm_up
        / (1.0 + warm_up)
        / (1.0 + cool_down)
    )
    new_history = out_ref[...]
    return (activations, loss, new_history)


# speedup vs baseline: 4.4734x; 1.2287x over previous
import jax
import jax.numpy as jnp
from jax import lax
from jax.experimental import pallas as pl
from jax.experimental.pallas import tpu as pltpu
from jax.experimental.pallas import tpu_sc as plsc

_WEIGHT = 0.1
_MOMENT = 0.9
_WARM_UP = 1.0 / 1000.0
_COOL_DOWN = 1.0 / 100000.0
_MAX_ITEMS = 1000000
_ITERATIONS = 500.0

_ROWS = _MAX_ITEMS + 1
_DIM = 128
_BATCH = 16384
_NVREG = _DIM // 16

_NC = 2
_NS = 16
_LANES = 16
_SENT = -1

_PER_TILE = _BATCH // _NS
_CHUNK = 128
_NCHUNK = _PER_TILE // _CHUNK

_B_PER_TILE = _BATCH // (_NC * _NS)
_B_NCHUNK = _B_PER_TILE // _CHUNK

_COPY_BLK = 8192


def _copy_body(src_ref, dst_ref):
    dst_ref[...] = src_ref[...]


def _pallas_copy(history):
    n_blocks = (_ROWS + _COPY_BLK - 1) // _COPY_BLK
    return pl.pallas_call(
        _copy_body,
        grid=(n_blocks,),
        in_specs=[pl.BlockSpec((_COPY_BLK, _DIM), lambda i: (i, 0))],
        out_specs=pl.BlockSpec((_COPY_BLK, _DIM), lambda i: (i, 0)),
        out_shape=jax.ShapeDtypeStruct((_ROWS, _DIM), jnp.float32),
    )(history)


def _sc_a_body(
    act_hbm,
    idx_hbm,
    hist_hbm,
    loss_hbm,
    lg_hbm,
    val_hbm,
    tag_sp,
    cnt_sm,
    sidx,
    gidx,
    lgidx,
    lslot,
    tgot,
    bval,
    rem_pos,
    act_c,
    row_c,
    val_c,
    b16,
    t16,
    tgix,
    lgix,
    asix,
    cvix,
    h16,
    a16,
    c16,
    v16,
    acc_v,
):
    c = lax.axis_index("c")
    s = lax.axis_index("s")
    slot0 = s * _PER_TILE
    lanes = lax.iota(jnp.int32, _LANES)

    pltpu.sync_copy(idx_hbm.at[pl.ds(slot0, _PER_TILE)], sidx)

    def _build(i, _):
        sl = pl.ds(i * _LANES, _LANES)
        v = sidx[sl]
        active = (v != 0) & (v < _MAX_ITEMS) & ((v & 1) == c)
        gidx[sl] = jnp.where(active, v, _SENT)
        bval[sl] = slot0 + i * _LANES + lanes
        return 0

    lax.fori_loop(0, _PER_TILE // _LANES, _build, 0)

    tag_ix = plsc.Indices(gidx, ignored_value=_SENT)
    pltpu.sync_copy(bval, tag_sp.at[tag_ix])
    plsc.subcore_barrier()
    pltpu.sync_copy(tag_sp.at[tag_ix], tgot)

    def _elect(i, off):
        sl = pl.ds(i * _LANES, _LANES)
        g = gidx[sl]
        b = bval[sl]
        act = g != _SENT
        lead = act & (tgot[sl] == b)
        lgidx[sl] = jnp.where(lead, g, _SENT)
        lslot[sl] = jnp.where(lead, b, _SENT)
        rem = jnp.where(act & ~lead, 1, 0)
        dst = jnp.maximum(off + plsc.cumsum(rem) - 1, 0)
        plsc.store_scatter(
            rem_pos, [dst], i * _LANES + lanes, mask=rem != 0
        )
        return off + jnp.sum(rem)

    n_rem = lax.fori_loop(0, _PER_TILE // _LANES, _elect, 0)

    pltpu.sync_copy(lgidx, lg_hbm.at[c, pl.ds(slot0, _PER_TILE)])

    acc = jnp.zeros((_LANES,), jnp.float32)
    for j in range(_NCHUNK):
        base = j * _CHUNK
        pltpu.sync_copy(act_hbm.at[pl.ds(slot0 + base, _CHUNK)], act_c)
        pltpu.sync_copy(
            hist_hbm.at[
                plsc.Indices(gidx.at[pl.ds(base, _CHUNK)], ignored_value=_SENT)
            ],
            row_c,
        )

        def _slots(i, a, base=base):
            isp = jnp.full((_LANES,), i, jnp.int32)
            am = (
                plsc.load_gather(gidx, [jnp.full((_LANES,), base, jnp.int32) + i])
                != _SENT
            )
            for r in range(_NVREG):
                cols = r * _LANES + lanes
                h = plsc.load_gather(row_c, [isp, cols])
                av = plsc.load_gather(act_c, [isp, cols])
                d = jnp.where(am, h - av, 0.0)
                a = a + d * d
                plsc.store_scatter(val_c, [isp, cols], h - (1.0 - _MOMENT) * d)
            return a

        acc = lax.fori_loop(0, _CHUNK, _slots, acc)
        pltpu.sync_copy(
            val_c,
            val_hbm.at[
                plsc.Indices(lslot.at[pl.ds(base, _CHUNK)], ignored_value=_SENT)
            ],
        )

    acc_v[...] = acc
    pltpu.sync_copy(acc_v, loss_hbm.at[c * _NS + s])

    def _count(local):
        plsc.subcore_barrier()

        @pl.when(s == 0)
        def _():
            cnt_sm[0] = 0

        plsc.subcore_barrier()
        plsc.fetch_and_add(cnt_sm, local, subcore_id=0)
        plsc.subcore_barrier()
        return plsc.fetch_and_add(cnt_sm, 0, subcore_id=0)

    def _round(carry):
        _, n_rem = carry
        ng = (n_rem + _LANES - 1) // _LANES

        def _scat(k, _):
            lm = k * _LANES + lanes < n_rem
            pos = jnp.where(lm, rem_pos[pl.ds(k * _LANES, _LANES)], 0)
            g = plsc.load_gather(gidx, [pos])
            tgix[...] = jnp.where(lm, g, _SENT)
            b16[...] = slot0 + pos
            pltpu.sync_copy(
                b16, tag_sp.at[plsc.Indices(tgix, ignored_value=_SENT)]
            )
            return 0

        lax.fori_loop(0, ng, _scat, 0)
        plsc.subcore_barrier()

        def _proc(k, off):
            lm = k * _LANES + lanes < n_rem
            pos = jnp.where(lm, rem_pos[pl.ds(k * _LANES, _LANES)], 0)
            g = plsc.load_gather(gidx, [pos])
            b = slot0 + pos
            tgix[...] = jnp.where(lm, g, _SENT)
            pltpu.sync_copy(
                tag_sp.at[plsc.Indices(tgix, ignored_value=_SENT)], t16
            )
            lead = lm & (t16[...] == b)
            winner = plsc.load_gather(tgot, [pos])
            lgix[...] = jnp.where(lead, g, _SENT)
            asix[...] = jnp.where(lead, b, _SENT)
            cvix[...] = jnp.where(lead, winner, _SENT)
            vix = plsc.Indices(cvix, ignored_value=_SENT)
            pltpu.sync_copy(
                hist_hbm.at[plsc.Indices(lgix, ignored_value=_SENT)], h16
            )
            pltpu.sync_copy(
                act_hbm.at[plsc.Indices(asix, ignored_value=_SENT)], a16
            )
            pltpu.sync_copy(val_hbm.at[vix], c16)

            def _rows(i, _):
                isp = jnp.full((_LANES,), i, jnp.int32)
                for r in range(_NVREG):
                    cols = r * _LANES + lanes
                    hh = plsc.load_gather(h16, [isp, cols])
                    aa = plsc.load_gather(a16, [isp, cols])
                    cc = plsc.load_gather(c16, [isp, cols])
                    plsc.store_scatter(
                        v16, [isp, cols], cc - (1.0 - _MOMENT) * (hh - aa)
                    )
                return 0

            lax.fori_loop(0, _LANES, _rows, 0)
            pltpu.sync_copy(v16, val_hbm.at[vix])

            keep = jnp.where(lm & ~lead, 1, 0)
            dst = jnp.maximum(off + plsc.cumsum(keep) - 1, 0)
            plsc.store_scatter(rem_pos, [dst], pos, mask=keep != 0)
            return off + jnp.sum(keep)

        n_rem = lax.fori_loop(0, ng, _proc, 0)
        return _count(n_rem), n_rem

    total = _count(n_rem)
    lax.while_loop(lambda t: t[0] > 0, _round, (total, n_rem))


def _sc_b_body(
    lg_hbm,
    val_hbm,
    out_ref,
    lg0,
    lg1,
    lgm,
    val_c,
):
    c = lax.axis_index("c")
    s = lax.axis_index("s")
    wid = c * _NS + s
    base0 = wid * _B_PER_TILE
    for j in range(_B_NCHUNK):
        base = base0 + j * _CHUNK
        pltpu.sync_copy(lg_hbm.at[0, pl.ds(base, _CHUNK)], lg0)
        pltpu.sync_copy(lg_hbm.at[1, pl.ds(base, _CHUNK)], lg1)

        def _merge(i, _):
            sl = pl.ds(i * _LANES, _LANES)
            a = lg0[sl]
            lgm[sl] = jnp.where(a != _SENT, a, lg1[sl])
            return 0

        lax.fori_loop(0, _CHUNK // _LANES, _merge, 0)
        pltpu.sync_copy(val_hbm.at[pl.ds(base, _CHUNK)], val_c)
        pltpu.sync_copy(
            val_c, out_ref.at[plsc.Indices(lgm, ignored_value=_SENT)]
        )


_sc_mesh = plsc.VectorSubcoreMesh(
    core_axis_name="c", subcore_axis_name="s", num_cores=_NC, num_subcores=_NS
)

_sc_a = pl.kernel(
    _sc_a_body,
    out_type=(
        jax.ShapeDtypeStruct((_NC * _NS, _LANES), jnp.float32),
        jax.ShapeDtypeStruct((_NC, _BATCH), jnp.int32),
        jax.ShapeDtypeStruct((_BATCH, _DIM), jnp.float32),
    ),
    mesh=_sc_mesh,
    compiler_params=pltpu.CompilerParams(needs_layout_passes=False),
    scratch_types=[
        pltpu.VMEM_SHARED((_ROWS,), jnp.int32),
        pltpu.SMEM((1,), jnp.int32),
        pltpu.VMEM((_PER_TILE,), jnp.int32),
        pltpu.VMEM((_PER_TILE,), jnp.int32),
        pltpu.VMEM((_PER_TILE,), jnp.int32),
        pltpu.VMEM((_PER_TILE,), jnp.int32),
        pltpu.VMEM((_PER_TILE,), jnp.int32),
        pltpu.VMEM((_PER_TILE,), jnp.int32),
        pltpu.VMEM((_PER_TILE + _LANES,), jnp.int32),
        pltpu.VMEM((_CHUNK, _DIM), jnp.float32),
        pltpu.VMEM((_CHUNK, _DIM), jnp.float32),
        pltpu.VMEM((_CHUNK, _DIM), jnp.float32),
        pltpu.VMEM((_LANES,), jnp.int32),
        pltpu.VMEM((_LANES,), jnp.int32),
        pltpu.VMEM((_LANES,), jnp.int32),
        pltpu.VMEM((_LANES,), jnp.int32),
        pltpu.VMEM((_LANES,), jnp.int32),
        pltpu.VMEM((_LANES,), jnp.int32),
        pltpu.VMEM((_LANES, _DIM), jnp.float32),
        pltpu.VMEM((_LANES, _DIM), jnp.float32),
        pltpu.VMEM((_LANES, _DIM), jnp.float32),
        pltpu.VMEM((_LANES, _DIM), jnp.float32),
        pltpu.VMEM((_LANES,), jnp.float32),
    ],
)

_sc_b = pl.kernel(
    _sc_b_body,
    out_type=(),
    mesh=_sc_mesh,
    compiler_params=pltpu.CompilerParams(needs_layout_passes=False),
    scratch_types=[
        pltpu.VMEM((_CHUNK,), jnp.int32),
        pltpu.VMEM((_CHUNK,), jnp.int32),
        pltpu.VMEM((_CHUNK,), jnp.int32),
        pltpu.VMEM((_CHUNK, _DIM), jnp.float32),
    ],
)


def kernel(activations, samples, history):
    idx = jnp.minimum(samples, _MAX_ITEMS).astype(jnp.int32).reshape(-1)
    copy = _pallas_copy(history)
    parts, lg, val = _sc_a(activations, idx, history)
    out_ref = jax.new_ref(copy)
    _sc_b(lg, val, out_ref)
    warm_up = _WARM_UP * _ITERATIONS
    cool_down = _COOL_DOWN * _ITERATIONS
    loss = (
        _WEIGHT
        * (jnp.sum(parts) / (_BATCH * _DIM))
        * warm_up
        / (1.0 + warm_up)
        / (1.0 + cool_down)
    )
    new_history = out_ref[...]
    return (activations, loss, new_history)
